# baseline (speedup 1.0000x reference)
import jax
import jax.numpy as jnp
from jax import lax
from jax.experimental import pallas as pl
from jax.experimental.pallas import tpu as pltpu

B, M, N = 64, 32, 8192
K = 16


def _select_kernel(p_ref, s_ref, eg_ref, vals_ref, idx_ref, sel_ref, s0p_ref,
                   g_vmem, dma_sem):
    copy = pltpu.make_async_copy(p_ref.at[:, 0, :], g_vmem, dma_sem)
    copy.start()

    sp = s_ref[0]
    sc = sp * eg_ref[...]
    sm = jnp.max(sc, axis=1, keepdims=True)
    iota2 = lax.broadcasted_iota(jnp.int32, (K, N), 1)
    sel = jnp.min(jnp.where(sc >= sm, iota2, N), axis=1)
    sel_ref[0, :] = sel
    s0p_ref[0, :] = jnp.sum(jnp.where(iota2 == sel[:, None], sp, 0.0), axis=1)

    copy.wait()
    x = g_vmem[...]
    iota = lax.broadcasted_iota(jnp.int32, (B, N), 1)
    vals = []
    idxs = []
    for _ in range(K):
        m = jnp.max(x, axis=1, keepdims=True)
        idx = jnp.min(jnp.where(x >= m, iota, N), axis=1, keepdims=True)
        vals.append(m)
        idxs.append(idx)
        x = jnp.where(iota == idx, -1.0, x)
    vals_ref[...] = jnp.maximum(jnp.concatenate(vals, axis=1), 1e-8)
    idx_ref[...] = jnp.concatenate(idxs, axis=1)


def _gather_kernel(sel_ref, p_ref, out_ref, g_vmem, dma_sem):
    copies = []
    for i in range(K):
        base = (sel_ref[i] // 128) * 128
        c = pltpu.make_async_copy(
            p_ref.at[:, K + i, pl.ds(base, 128)], g_vmem.at[i], dma_sem)
        c.start()
        copies.append(c)
    for c in copies:
        c.wait()
    lane = lax.broadcasted_iota(jnp.int32, (B, 128), 1)
    for i in range(K):
        r = sel_ref[i] % 128
        v = jnp.sum(jnp.where(lane == r, g_vmem[i], 0.0), axis=1)
        out_ref[i, :] = jnp.maximum(v, 1e-8)


@jax.jit
def kernel(probs):
    eg = jnp.exp(jax.random.gumbel(jax.random.key(42), (K, N), jnp.float32))

    vals, idx, sel2d, s0p = pl.pallas_call(
        _select_kernel,
        grid=(1,),
        in_specs=[
            pl.BlockSpec(memory_space=pl.ANY),
            pl.BlockSpec((1, K, N), lambda i: (0, 1, 0)),
            pl.BlockSpec((K, N), lambda i: (0, 0)),
        ],
        out_specs=[
            pl.BlockSpec((B, K), lambda i: (0, 0)),
            pl.BlockSpec((B, K), lambda i: (0, 0)),
            pl.BlockSpec((1, K), lambda i: (0, 0)),
            pl.BlockSpec((1, K), lambda i: (0, 0)),
        ],
        out_shape=[
            jax.ShapeDtypeStruct((B, K), jnp.float32),
            jax.ShapeDtypeStruct((B, K), jnp.int32),
            jax.ShapeDtypeStruct((1, K), jnp.int32),
            jax.ShapeDtypeStruct((1, K), jnp.float32),
        ],
        scratch_shapes=[
            pltpu.VMEM((B, N), jnp.float32),
            pltpu.SemaphoreType.DMA,
        ],
    )(probs, probs, eg)

    sel = sel2d[0]

    grid_spec = pltpu.PrefetchScalarGridSpec(
        num_scalar_prefetch=1,
        grid=(1,),
        in_specs=[pl.BlockSpec(memory_space=pl.ANY)],
        out_specs=pl.BlockSpec((K, B), lambda g, sr: (0, 0)),
        scratch_shapes=[
            pltpu.VMEM((K, B, 128), jnp.float32),
            pltpu.SemaphoreType.DMA,
        ],
    )
    sprobs = pl.pallas_call(
        _gather_kernel,
        grid_spec=grid_spec,
        out_shape=jax.ShapeDtypeStruct((K, B), jnp.float32),
    )(sel, probs)

    selected = jnp.concatenate(
        [idx, jnp.broadcast_to(sel[None, :], (B, K))], axis=1)
    prob = jnp.concatenate([vals, sprobs.T], axis=1)
    return selected, prob
